# baseline (speedup 1.0000x reference)
import functools

import jax
import jax.numpy as jnp
from jax import lax
from jax.experimental import pallas as pl
from jax.experimental.pallas import tpu as pltpu, tpu_sc as plsc

EMB_DIM = 32
LANES = 16


def _mf_body(b_per_w, nc, p1_hbm, p2_hbm, table_hbm, out_hbm,
             idx1_v, idx2_v, rows1_v, rows2_v, out_v, sem1, sem2):
    wid = lax.axis_index("s") * nc + lax.axis_index("c")
    base = wid * b_per_w

    pltpu.sync_copy(p1_hbm.at[pl.ds(base, b_per_w)], idx1_v)
    pltpu.sync_copy(p2_hbm.at[pl.ds(base, b_per_w)], idx2_v)
    cp1 = pltpu.async_copy(table_hbm.at[idx1_v], rows1_v, sem1)
    cp2 = pltpu.async_copy(table_hbm.at[idx2_v], rows2_v, sem2)
    cp1.wait()
    cp2.wait()

    iota16 = lax.iota(jnp.int32, LANES)

    def group_body(g, carry):
        row_ids = g * LANES + iota16
        acc = jnp.zeros((LANES,), jnp.float32)
        for d in range(EMB_DIM):
            col = jnp.full((LANES,), d, jnp.int32)
            a = plsc.load_gather(rows1_v, [row_ids, col])
            b = plsc.load_gather(rows2_v, [row_ids, col])
            acc = acc + a * b
        out_v[pl.ds(g * LANES, LANES)] = 1.0 / (1.0 + jnp.exp(-acc))
        return carry

    lax.fori_loop(0, b_per_w // LANES, group_body, 0)
    pltpu.sync_copy(out_v, out_hbm.at[pl.ds(base, b_per_w)])


def kernel(product1, product2, embedding_weight):
    batch = product1.shape[0]
    info = plsc.get_sparse_core_info()
    nc, ns = info.num_cores, info.num_subcores
    nw = nc * ns
    b_per_w = batch // nw
    mesh = plsc.VectorSubcoreMesh(core_axis_name="c", subcore_axis_name="s")
    run = pl.kernel(
        functools.partial(_mf_body, b_per_w, nc),
        out_type=jax.ShapeDtypeStruct((batch,), jnp.float32),
        mesh=mesh,
        scratch_types=[
            pltpu.VMEM((b_per_w,), jnp.int32),
            pltpu.VMEM((b_per_w,), jnp.int32),
            pltpu.VMEM((b_per_w, EMB_DIM), jnp.float32),
            pltpu.VMEM((b_per_w, EMB_DIM), jnp.float32),
            pltpu.VMEM((b_per_w,), jnp.float32),
            pltpu.SemaphoreType.DMA,
            pltpu.SemaphoreType.DMA,
        ],
        compiler_params=pltpu.CompilerParams(needs_layout_passes=False,
                                             use_tc_tiling_on_sc=False),
    )
    return run(product1.astype(jnp.int32), product2.astype(jnp.int32),
               embedding_weight)
